# Initial kernel scaffold; baseline (speedup 1.0000x reference)
import jax
import jax.numpy as jnp
from jax.experimental import pallas as pl


def kernel(x, edge_index, W1, b1, W2, b2):
    raise NotImplementedError("write your pallas kernel here")


# speedup vs baseline: 12.3505x; 12.3505x over previous
import functools

import jax
import jax.numpy as jnp
from jax import lax
from jax.experimental import pallas as pl
from jax.experimental.pallas import tpu as pltpu
from jax.experimental.pallas import tpu_sc as plsc

N = 10000
D = 128
C = 16
E = 320000
NC = 2
NS = 16
NW = NC * NS
EPW = E // NW
CHUNK = 80
NCHUNK = EPW // CHUNK
RPT = 632
RPT_LAST = N - (NS - 1) * RPT
BLK = 1024
GRID = (N + BLK - 1) // BLK

_mesh = plsc.VectorSubcoreMesh(core_axis_name="c", subcore_axis_name="s")
_f32 = jnp.float32


@functools.partial(
    pl.kernel,
    out_type=[jax.ShapeDtypeStruct((NC, N), _f32),
              jax.ShapeDtypeStruct((NC, N), _f32)],
    mesh=_mesh,
    scratch_types=[
        pltpu.VMEM((NCHUNK, CHUNK), jnp.int32),
        pltpu.VMEM((NCHUNK, CHUNK), jnp.int32),
        pltpu.VMEM((CHUNK,), _f32),
        pltpu.VMEM_SHARED((N,), _f32),
        pltpu.VMEM_SHARED((N,), _f32),
    ],
    compiler_params=pltpu.CompilerParams(needs_layout_passes=False),
)
def _deg_kernel(src_hbm, dst_hbm, z1_hbm, do_hbm, di_hbm,
                src_v, dst_v, ones_v, do_sh, di_sh):
    cid = lax.axis_index("c")
    sid = lax.axis_index("s")
    wid = cid * NS + sid
    pltpu.sync_copy(src_hbm.at[wid], src_v)
    pltpu.sync_copy(dst_hbm.at[wid], dst_v)

    def _init_ones(k, carry):
        ones_v[pl.ds(k * 16, 16)] = jnp.ones((16,), _f32)
        return carry
    lax.fori_loop(0, CHUNK // 16, _init_ones, 0)

    @pl.when(sid == 0)
    def _():
        pltpu.sync_copy(z1_hbm, do_sh)
        pltpu.sync_copy(z1_hbm, di_sh)
    plsc.subcore_barrier()

    def _step(j, carry):
        pltpu.sync_copy(ones_v, do_sh.at[src_v.at[j]], add=True)
        pltpu.sync_copy(ones_v, di_sh.at[dst_v.at[j]], add=True)
        return carry
    lax.fori_loop(0, NCHUNK, _step, 0)
    plsc.subcore_barrier()

    @pl.when(sid == 0)
    def _():
        pltpu.sync_copy(do_sh, do_hbm.at[cid])
        pltpu.sync_copy(di_sh, di_hbm.at[cid])


@functools.partial(
    pl.kernel,
    out_type=[jax.ShapeDtypeStruct((NC, N, D), _f32),
              jax.ShapeDtypeStruct((NC, N), _f32)],
    mesh=_mesh,
    scratch_types=[
        pltpu.VMEM((NCHUNK, CHUNK), jnp.int32),
        pltpu.VMEM((NCHUNK, CHUNK), jnp.int32),
        pltpu.VMEM((CHUNK, D), _f32),
        pltpu.VMEM((CHUNK,), _f32),
        pltpu.VMEM_SHARED((N, D), _f32),
        pltpu.VMEM_SHARED((N,), _f32),
        pltpu.SemaphoreType.DMA,
        pltpu.SemaphoreType.DMA,
    ],
    compiler_params=pltpu.CompilerParams(needs_layout_passes=False),
)
def _agg_kernel(src_hbm, dst_hbm, y_hbm, nd_hbm, z1_hbm, z2_hbm,
                agg_hbm, c_hbm,
                src_v, dst_v, rows_v, cupd_v, agg_sh, c_sh, sem, sem2):
    cid = lax.axis_index("c")
    sid = lax.axis_index("s")
    wid = cid * NS + sid
    pltpu.sync_copy(src_hbm.at[wid], src_v)
    pltpu.sync_copy(dst_hbm.at[wid], dst_v)
    @pl.when(sid < NS - 1)
    def _():
        pltpu.sync_copy(z2_hbm.at[pl.ds(sid * RPT, RPT)],
                        agg_sh.at[pl.ds(sid * RPT, RPT)])

    @pl.when(sid == NS - 1)
    def _():
        pltpu.sync_copy(z2_hbm.at[pl.ds((NS - 1) * RPT, RPT_LAST)],
                        agg_sh.at[pl.ds((NS - 1) * RPT, RPT_LAST)])

    @pl.when(sid == 0)
    def _():
        pltpu.sync_copy(z1_hbm, c_sh)
    plsc.subcore_barrier()

    def _step(j, carry):
        cp = pltpu.async_copy(y_hbm.at[src_v.at[j]], rows_v, sem)
        cp2 = pltpu.async_copy(nd_hbm.at[dst_v.at[j]], cupd_v, sem2)
        cp.wait()
        pltpu.sync_copy(rows_v, agg_sh.at[dst_v.at[j]], add=True)
        cp2.wait()
        pltpu.sync_copy(cupd_v, c_sh.at[src_v.at[j]], add=True)
        return carry
    lax.fori_loop(0, NCHUNK, _step, 0)
    plsc.subcore_barrier()

    @pl.when(sid < NS - 1)
    def _():
        pltpu.sync_copy(agg_sh.at[pl.ds(sid * RPT, RPT)],
                        agg_hbm.at[cid, pl.ds(sid * RPT, RPT)])

    @pl.when(sid == NS - 1)
    def _():
        pltpu.sync_copy(agg_sh.at[pl.ds((NS - 1) * RPT, RPT_LAST)],
                        agg_hbm.at[cid, pl.ds((NS - 1) * RPT, RPT_LAST)])

    @pl.when(sid == 0)
    def _():
        pltpu.sync_copy(c_sh, c_hbm.at[cid])


def _norms_body(dop_ref, dip_ref, ns_ref, nd_ref):
    do = dop_ref[0:1, :] + dop_ref[1:2, :]
    di = dip_ref[0:1, :] + dip_ref[1:2, :]
    ns_ref[...] = lax.rsqrt(jnp.maximum(do, 1.0))
    nd_ref[...] = lax.rsqrt(jnp.maximum(di, 1.0))


def _mm_body(x_ref, w_ref, ns_ref, y_ref):
    y_ref[...] = jnp.dot(x_ref[...], w_ref[...],
                         preferred_element_type=_f32) * ns_ref[...]


def _fin_body(agg_ref, nd_ref, ns_ref, cp_ref, b1_ref, w2_ref, b2_ref,
              out_ref, acc_ref):
    i = pl.program_id(0)

    @pl.when(i == 0)
    def _():
        acc_ref[...] = jnp.zeros_like(acc_ref)

    agg = agg_ref[0] + agg_ref[1]
    h1 = jnp.maximum(agg * nd_ref[...] + b1_ref[...], 0.0)
    c = ns_ref[...] * (cp_ref[0] + cp_ref[1])
    rows = i * BLK + lax.broadcasted_iota(jnp.int32, (BLK, 1), 0)
    contrib = jnp.where(rows < N, h1 * c, 0.0)
    acc_ref[...] += jnp.sum(contrib, axis=0, keepdims=True)

    @pl.when(i == pl.num_programs(0) - 1)
    def _():
        v = acc_ref[...] * (1.0 / N)
        out_ref[...] = jnp.dot(v, w2_ref[...],
                               preferred_element_type=_f32) + b2_ref[...]


def kernel(x, edge_index, W1, b1, W2, b2):
    src = edge_index[0].astype(jnp.int32).reshape(NW, NCHUNK, CHUNK)
    dst = edge_index[1].astype(jnp.int32).reshape(NW, NCHUNK, CHUNK)
    z1 = jnp.zeros((N,), _f32)
    z2 = jnp.zeros((N, D), _f32)

    do_p, di_p = _deg_kernel(src, dst, z1)

    ns_row, nd_row = pl.pallas_call(
        _norms_body,
        out_shape=[jax.ShapeDtypeStruct((1, N), _f32),
                   jax.ShapeDtypeStruct((1, N), _f32)],
    )(do_p, di_p)
    ns_col = ns_row.reshape(N, 1)
    nd_col = nd_row.reshape(N, 1)
    nd_flat = nd_row.reshape(N)

    y = pl.pallas_call(
        _mm_body,
        grid=(GRID,),
        in_specs=[
            pl.BlockSpec((BLK, D), lambda i: (i, 0)),
            pl.BlockSpec((D, D), lambda i: (0, 0)),
            pl.BlockSpec((BLK, 1), lambda i: (i, 0)),
        ],
        out_specs=pl.BlockSpec((BLK, D), lambda i: (i, 0)),
        out_shape=jax.ShapeDtypeStruct((N, D), _f32),
    )(x, W1, ns_col)

    agg_p, c_p = _agg_kernel(src, dst, y, nd_flat, z1, z2)
    c_p3 = c_p.reshape(NC, N, 1)

    out = pl.pallas_call(
        _fin_body,
        grid=(GRID,),
        in_specs=[
            pl.BlockSpec((NC, BLK, D), lambda i: (0, i, 0)),
            pl.BlockSpec((BLK, 1), lambda i: (i, 0)),
            pl.BlockSpec((BLK, 1), lambda i: (i, 0)),
            pl.BlockSpec((NC, BLK, 1), lambda i: (0, i, 0)),
            pl.BlockSpec((1, D), lambda i: (0, 0)),
            pl.BlockSpec((D, C), lambda i: (0, 0)),
            pl.BlockSpec((1, C), lambda i: (0, 0)),
        ],
        out_specs=pl.BlockSpec((1, C), lambda i: (0, 0)),
        out_shape=jax.ShapeDtypeStruct((1, C), _f32),
        scratch_shapes=[pltpu.VMEM((1, D), _f32)],
    )(agg_p, nd_col, ns_col, c_p3, b1.reshape(1, D), W2, b2.reshape(1, C))

    return out.reshape(C)
